# Initial kernel scaffold; baseline (speedup 1.0000x reference)
import jax
import jax.numpy as jnp
from jax.experimental import pallas as pl


def kernel(x, edge_index, edge_attr, w1, b1, root1, bias1, gamma1, beta1, w2, b2, root2, bias2, gamma2, beta2):
    raise NotImplementedError("write your pallas kernel here")


# speedup vs baseline: 3.8656x; 3.8656x over previous
import functools

import jax
import jax.numpy as jnp
from jax import lax
from jax.experimental import pallas as pl
from jax.experimental.pallas import tpu as pltpu
from jax.experimental.pallas import tpu_sc as plsc

F32 = jnp.float32

_NC, _NS = 2, 16
_NW = _NC * _NS
_CH = 128
_CW = 16
_SC_PARAMS = pltpu.CompilerParams(use_tc_tiling_on_sc=False)


def _sc_gather(table, idx3):
    nw, k, ch = idx3.shape
    d = table.shape[1]
    epw = k * ch
    e_pad = nw * epw
    mesh = plsc.VectorSubcoreMesh(core_axis_name="c", subcore_axis_name="s")

    @functools.partial(
        pl.kernel,
        out_type=jax.ShapeDtypeStruct((e_pad, d), F32),
        mesh=mesh,
        scratch_types=[
            pltpu.VMEM((k, ch), jnp.int32),
            pltpu.VMEM((epw, d), F32),
            pltpu.SemaphoreType.DMA,
        ],
        compiler_params=_SC_PARAMS,
    )
    def kern(table_hbm, idx_hbm, out_hbm, idx_v, rows_v, sem):
        wid = lax.axis_index("s") * _NC + lax.axis_index("c")
        base = pl.multiple_of(wid * epw, ch)
        pltpu.sync_copy(idx_hbm.at[wid], idx_v)
        cps = [
            pltpu.async_copy(table_hbm.at[idx_v.at[j]],
                             rows_v.at[pl.ds(j * ch, ch)], sem)
            for j in range(k)
        ]
        for c in cps:
            c.wait()
        pltpu.sync_copy(rows_v, out_hbm.at[pl.ds(base, epw)])

    return kern(table, idx3)


def _sc_scatter(msg, dst3, zeros_d, zeros_c=None, ones_c=None):
    nw, k, ch = dst3.shape
    d = msg.shape[1]
    n_acc = zeros_d.shape[0]
    rpt = n_acc // _NS
    epw = k * ch
    with_counts = zeros_c is not None
    mesh = plsc.VectorSubcoreMesh(core_axis_name="c", subcore_axis_name="s")

    out_type = [jax.ShapeDtypeStruct((_NC * n_acc, d), F32)]
    scratch = [
        pltpu.VMEM((k, ch), jnp.int32),
        pltpu.VMEM((ch, d), F32),
        pltpu.VMEM_SHARED((n_acc, d), F32),
    ]
    if with_counts:
        out_type.append(jax.ShapeDtypeStruct((_NC * n_acc, _CW), F32))
        scratch += [
            pltpu.VMEM((ch, _CW), F32),
            pltpu.VMEM_SHARED((n_acc, _CW), F32),
        ]

    def kern(msg_hbm, dst_hbm, zd_hbm, *rest):
        if with_counts:
            (zc_hbm, ones_hbm, s_out, c_out,
             idx_v, msg_v, acc, ones_v, accc) = rest
        else:
            zc_hbm = ones_hbm = c_out = ones_v = accc = None
            (s_out, idx_v, msg_v, acc) = rest
        cid = lax.axis_index("c")
        sid = lax.axis_index("s")
        wid = sid * _NC + cid
        base = pl.multiple_of(wid * epw, ch)
        stripe = pl.multiple_of(sid * rpt, 8)
        pltpu.sync_copy(dst_hbm.at[wid], idx_v)
        pltpu.sync_copy(zd_hbm.at[pl.ds(stripe, rpt)],
                        acc.at[pl.ds(stripe, rpt)])
        if with_counts:
            pltpu.sync_copy(ones_hbm, ones_v)
            pltpu.sync_copy(zc_hbm.at[pl.ds(stripe, rpt)],
                            accc.at[pl.ds(stripe, rpt)])
        plsc.subcore_barrier()
        for j in range(k):
            pltpu.sync_copy(msg_hbm.at[pl.ds(base + j * ch, ch)], msg_v)
            pltpu.sync_copy(msg_v, acc.at[idx_v.at[j]], add=True)
            if with_counts:
                pltpu.sync_copy(ones_v, accc.at[idx_v.at[j]], add=True)
        plsc.subcore_barrier()
        obase = pl.multiple_of(cid * n_acc + stripe, 8)
        pltpu.sync_copy(acc.at[pl.ds(stripe, rpt)],
                        s_out.at[pl.ds(obase, rpt)])
        if with_counts:
            pltpu.sync_copy(accc.at[pl.ds(stripe, rpt)],
                            c_out.at[pl.ds(obase, rpt)])

    f = pl.kernel(kern, out_type=tuple(out_type), mesh=mesh,
                  scratch_types=scratch, compiler_params=_SC_PARAMS)
    args = (msg, dst3, zeros_d) + ((zeros_c, ones_c) if with_counts else ())
    res = f(*args)
    return res if with_counts else (res[0] if isinstance(res, (tuple, list)) else res)


def _tc_messages(xj, attr, w_p, b_p, d_in, be=512):
    e_pad, dj = xj.shape
    d_out = w_p.shape[1]

    def body(xj_ref, a_ref, w_ref, b_ref, o_ref):
        a = a_ref[...]
        xjb = xj_ref[...]
        acc = jnp.zeros((be, d_out), F32)
        for i in range(d_in):
            ew = jnp.maximum(a * w_ref[i:i + 1, :] + b_ref[i:i + 1, :], 0.0)
            acc = acc + xjb[:, i:i + 1] * ew
        o_ref[...] = acc

    return pl.pallas_call(
        body,
        grid=(e_pad // be,),
        in_specs=[
            pl.BlockSpec((be, dj), lambda i: (i, 0)),
            pl.BlockSpec((be, 1), lambda i: (i, 0)),
            pl.BlockSpec(w_p.shape, lambda i: (0, 0)),
            pl.BlockSpec(b_p.shape, lambda i: (0, 0)),
        ],
        out_specs=pl.BlockSpec((be, d_out), lambda i: (i, 0)),
        out_shape=jax.ShapeDtypeStruct((e_pad, d_out), F32),
    )(xj, attr, w_p, b_p)


def _tc_combine(s_cat, cnt_cat, xin, root_p, bias, gamma, beta, n, n_acc,
                final_gram):
    d = s_cat.shape[1]

    def body(s_ref, c_ref, x_ref, r_ref, b_ref, g_ref, be_ref, o_ref):
        s = s_ref[...]
        agg_sum = s[:n] + s[n_acc:n_acc + n]
        cc = c_ref[...]
        cnt = cc[:n, 0:1] + cc[n_acc:n_acc + n, 0:1]
        agg = agg_sum / jnp.maximum(cnt, 1.0)
        h = agg + jnp.dot(x_ref[...], r_ref[...],
                          preferred_element_type=F32) + b_ref[...]
        m = jnp.mean(h, axis=0, keepdims=True)
        v = jnp.mean((h - m) ** 2, axis=0, keepdims=True)
        z = (h - m) * lax.rsqrt(v + 1e-3) * g_ref[...] + be_ref[...]
        x_out = jax.nn.sigmoid(z)
        if final_gram:
            o_ref[...] = lax.dot_general(
                x_out, x_out, (((0,), (0,)), ((), ())),
                preferred_element_type=F32)
        else:
            o_ref[...] = x_out

    out_shape = (d, d) if final_gram else (n, d)
    return pl.pallas_call(
        body,
        out_shape=jax.ShapeDtypeStruct(out_shape, F32),
    )(s_cat, cnt_cat, xin, root_p, bias, gamma, beta)


def kernel(x, edge_index, edge_attr, w1, b1, root1, bias1, gamma1, beta1,
           w2, b2, root2, bias2, gamma2, beta2):
    n, d_in = x.shape
    e = edge_index.shape[1]
    d_mid = root1.shape[1]
    d_out = root2.shape[1]
    dp = 48
    wr = 40
    k = -(-e // (_NW * _CH))
    e_pad = _NW * _CH * k
    pad_e = e_pad - e
    n_acc = -(-(n + 1) // 128) * 128

    src = edge_index[0].astype(jnp.int32)
    dst = edge_index[1].astype(jnp.int32)
    src_p = jnp.concatenate(
        [src, jnp.zeros((pad_e,), jnp.int32)]).reshape(_NW, k, _CH)
    dst_p = jnp.concatenate(
        [dst, jnp.full((pad_e,), n, jnp.int32)]).reshape(_NW, k, _CH)
    attr_p = jnp.pad(edge_attr.astype(F32), ((0, pad_e), (0, 0)))
    xp = jnp.pad(x, ((0, 0), (0, dp - d_in)))

    w1m = jnp.pad(w1[:, 0].reshape(d_in, d_mid),
                  ((0, wr - d_in), (0, dp - d_mid)))
    b1m = jnp.pad(b1.reshape(d_in, d_mid), ((0, wr - d_in), (0, dp - d_mid)))
    w2m = jnp.pad(w2[:, 0].reshape(d_mid, d_out), ((0, wr - d_mid), (0, 0)))
    b2m = jnp.pad(b2.reshape(d_mid, d_out), ((0, wr - d_mid), (0, 0)))
    root1p = jnp.pad(root1, ((0, dp - d_in), (0, dp - d_mid)))
    root2p = jnp.pad(root2, ((0, dp - d_mid), (0, 0)))
    bias1p = jnp.pad(bias1, (0, dp - d_mid)).reshape(1, dp)
    gamma1p = jnp.pad(gamma1, (0, dp - d_mid)).reshape(1, dp)
    beta1p = jnp.pad(beta1, (0, dp - d_mid)).reshape(1, dp)
    bias2p = bias2.reshape(1, d_out)
    gamma2p = gamma2.reshape(1, d_out)
    beta2p = beta2.reshape(1, d_out)

    zeros_d1 = jnp.zeros((n_acc, dp), F32)
    zeros_cw = jnp.zeros((n_acc, _CW), F32)
    ones_cw = jnp.ones((_CH, _CW), F32)
    zeros_d2 = jnp.zeros((n_acc, d_out), F32)

    xj1 = _sc_gather(xp, src_p)
    msg1 = _tc_messages(xj1, attr_p, w1m, b1m, d_in)
    s1, cnt = _sc_scatter(msg1, dst_p, zeros_d1, zeros_cw, ones_cw)
    x1 = _tc_combine(s1, cnt, xp, root1p, bias1p, gamma1p, beta1p,
                     n, n_acc, False)
    xj2 = _sc_gather(x1, src_p)
    msg2 = _tc_messages(xj2, attr_p, w2m, b2m, d_mid)
    s2 = _sc_scatter(msg2, dst_p, zeros_d2)
    x3 = _tc_combine(s2, cnt, x1, root2p, bias2p, gamma2p, beta2p,
                     n, n_acc, True)
    return x3


# speedup vs baseline: 5.2944x; 1.3696x over previous
import functools

import jax
import jax.numpy as jnp
from jax import lax
from jax.experimental import pallas as pl
from jax.experimental.pallas import tpu as pltpu
from jax.experimental.pallas import tpu_sc as plsc

F32 = jnp.float32

_NC, _NS = 2, 16
_NW = _NC * _NS
_CH = 128
_CW = 16
_SC_PARAMS = pltpu.CompilerParams(use_tc_tiling_on_sc=False)


def _sc_gather(table, idx3):
    nw, k, ch = idx3.shape
    d = table.shape[1]
    epw = k * ch
    e_pad = nw * epw
    mesh = plsc.VectorSubcoreMesh(core_axis_name="c", subcore_axis_name="s")

    @functools.partial(
        pl.kernel,
        out_type=jax.ShapeDtypeStruct((e_pad, d), F32),
        mesh=mesh,
        scratch_types=[
            pltpu.VMEM((k, ch), jnp.int32),
            pltpu.VMEM((epw, d), F32),
            pltpu.SemaphoreType.DMA,
        ],
        compiler_params=_SC_PARAMS,
    )
    def kern(table_hbm, idx_hbm, out_hbm, idx_v, rows_v, sem):
        wid = lax.axis_index("s") * _NC + lax.axis_index("c")
        base = pl.multiple_of(wid * epw, ch)
        pltpu.sync_copy(idx_hbm.at[wid], idx_v)
        cps = [
            pltpu.async_copy(table_hbm.at[idx_v.at[j]],
                             rows_v.at[pl.ds(j * ch, ch)], sem)
            for j in range(k)
        ]
        for c in cps:
            c.wait()
        pltpu.sync_copy(rows_v, out_hbm.at[pl.ds(base, epw)])

    return kern(table, idx3)


def _sc_scatter(msg, dst3, zeros_d, zeros_c=None, ones_c=None):
    nw, k, ch = dst3.shape
    d = msg.shape[1]
    n_acc = zeros_d.shape[0]
    rpt = n_acc // _NS
    epw = k * ch
    with_counts = zeros_c is not None
    mesh = plsc.VectorSubcoreMesh(core_axis_name="c", subcore_axis_name="s")

    out_type = [jax.ShapeDtypeStruct((_NC * n_acc, d), F32)]
    scratch = [
        pltpu.VMEM((k, ch), jnp.int32),
        pltpu.VMEM((ch, d), F32),
        pltpu.VMEM_SHARED((n_acc, d), F32),
    ]
    if with_counts:
        out_type.append(jax.ShapeDtypeStruct((_NC * n_acc, _CW), F32))
        scratch += [
            pltpu.VMEM((ch, _CW), F32),
            pltpu.VMEM_SHARED((n_acc, _CW), F32),
        ]

    def kern(msg_hbm, dst_hbm, zd_hbm, *rest):
        if with_counts:
            (zc_hbm, ones_hbm, s_out, c_out,
             idx_v, msg_v, acc, ones_v, accc) = rest
        else:
            zc_hbm = ones_hbm = c_out = ones_v = accc = None
            (s_out, idx_v, msg_v, acc) = rest
        cid = lax.axis_index("c")
        sid = lax.axis_index("s")
        wid = sid * _NC + cid
        base = pl.multiple_of(wid * epw, ch)
        stripe = pl.multiple_of(sid * rpt, 8)
        pltpu.sync_copy(dst_hbm.at[wid], idx_v)
        pltpu.sync_copy(zd_hbm.at[pl.ds(stripe, rpt)],
                        acc.at[pl.ds(stripe, rpt)])
        if with_counts:
            pltpu.sync_copy(ones_hbm, ones_v)
            pltpu.sync_copy(zc_hbm.at[pl.ds(stripe, rpt)],
                            accc.at[pl.ds(stripe, rpt)])
        plsc.subcore_barrier()
        for j in range(k):
            pltpu.sync_copy(msg_hbm.at[pl.ds(base + j * ch, ch)], msg_v)
            pltpu.sync_copy(msg_v, acc.at[idx_v.at[j]], add=True)
            if with_counts:
                pltpu.sync_copy(ones_v, accc.at[idx_v.at[j]], add=True)
        plsc.subcore_barrier()
        obase = pl.multiple_of(cid * n_acc + stripe, 8)
        pltpu.sync_copy(acc.at[pl.ds(stripe, rpt)],
                        s_out.at[pl.ds(obase, rpt)])
        if with_counts:
            pltpu.sync_copy(accc.at[pl.ds(stripe, rpt)],
                            c_out.at[pl.ds(obase, rpt)])

    f = pl.kernel(kern, out_type=tuple(out_type), mesh=mesh,
                  scratch_types=scratch, compiler_params=_SC_PARAMS)
    args = (msg, dst3, zeros_d) + ((zeros_c, ones_c) if with_counts else ())
    res = f(*args)
    return res if with_counts else (res[0] if isinstance(res, (tuple, list)) else res)


def _tc_messages(xj, attr, w_p, b_p, d_in, be=1024):
    e_pad, dj = xj.shape
    d_out = w_p.shape[1]

    bf16 = jnp.bfloat16

    def body(xj_ref, a_ref, w_ref, b_ref, o_ref):
        a = a_ref[...].astype(bf16)
        xjb = xj_ref[...].astype(bf16)
        w = w_ref[...].astype(bf16)
        b = b_ref[...].astype(bf16)
        acc = jnp.zeros((be, d_out), bf16)
        for i in range(d_in):
            ew = jnp.maximum(a * w[i:i + 1, :] + b[i:i + 1, :], bf16(0.0))
            acc = acc + xjb[:, i:i + 1] * ew
        o_ref[...] = acc.astype(F32)

    return pl.pallas_call(
        body,
        grid=(e_pad // be,),
        in_specs=[
            pl.BlockSpec((be, dj), lambda i: (i, 0)),
            pl.BlockSpec((be, 1), lambda i: (i, 0)),
            pl.BlockSpec(w_p.shape, lambda i: (0, 0)),
            pl.BlockSpec(b_p.shape, lambda i: (0, 0)),
        ],
        out_specs=pl.BlockSpec((be, d_out), lambda i: (i, 0)),
        out_shape=jax.ShapeDtypeStruct((e_pad, d_out), F32),
    )(xj, attr, w_p, b_p)


def _tc_combine(s_cat, cnt_cat, xin, root_p, bias, gamma, beta, n, n_acc,
                final_gram):
    d = s_cat.shape[1]

    def body(s_ref, c_ref, x_ref, r_ref, b_ref, g_ref, be_ref, o_ref):
        s = s_ref[...]
        agg_sum = s[:n] + s[n_acc:n_acc + n]
        cc = c_ref[...]
        cnt = cc[:n, 0:1] + cc[n_acc:n_acc + n, 0:1]
        agg = agg_sum / jnp.maximum(cnt, 1.0)
        h = agg + jnp.dot(x_ref[...], r_ref[...],
                          preferred_element_type=F32) + b_ref[...]
        m = jnp.mean(h, axis=0, keepdims=True)
        v = jnp.mean((h - m) ** 2, axis=0, keepdims=True)
        z = (h - m) * lax.rsqrt(v + 1e-3) * g_ref[...] + be_ref[...]
        x_out = jax.nn.sigmoid(z)
        if final_gram:
            o_ref[...] = lax.dot_general(
                x_out, x_out, (((0,), (0,)), ((), ())),
                preferred_element_type=F32)
        else:
            o_ref[...] = x_out

    out_shape = (d, d) if final_gram else (n, d)
    return pl.pallas_call(
        body,
        out_shape=jax.ShapeDtypeStruct(out_shape, F32),
    )(s_cat, cnt_cat, xin, root_p, bias, gamma, beta)


def kernel(x, edge_index, edge_attr, w1, b1, root1, bias1, gamma1, beta1,
           w2, b2, root2, bias2, gamma2, beta2):
    n, d_in = x.shape
    e = edge_index.shape[1]
    d_mid = root1.shape[1]
    d_out = root2.shape[1]
    dp = 48
    wr = 40
    k = -(-e // (_NW * _CH))
    e_pad = _NW * _CH * k
    pad_e = e_pad - e
    n_acc = -(-(n + 1) // 128) * 128

    src = edge_index[0].astype(jnp.int32)
    dst = edge_index[1].astype(jnp.int32)
    src_p = jnp.concatenate(
        [src, jnp.zeros((pad_e,), jnp.int32)]).reshape(_NW, k, _CH)
    dst_p = jnp.concatenate(
        [dst, jnp.full((pad_e,), n, jnp.int32)]).reshape(_NW, k, _CH)
    attr_p = jnp.pad(edge_attr.astype(F32), ((0, pad_e), (0, 0)))
    xp = jnp.pad(x, ((0, 0), (0, dp - d_in)))

    w1m = jnp.pad(w1[:, 0].reshape(d_in, d_mid),
                  ((0, wr - d_in), (0, dp - d_mid)))
    b1m = jnp.pad(b1.reshape(d_in, d_mid), ((0, wr - d_in), (0, dp - d_mid)))
    w2m = jnp.pad(w2[:, 0].reshape(d_mid, d_out), ((0, wr - d_mid), (0, 0)))
    b2m = jnp.pad(b2.reshape(d_mid, d_out), ((0, wr - d_mid), (0, 0)))
    root1p = jnp.pad(root1, ((0, dp - d_in), (0, dp - d_mid)))
    root2p = jnp.pad(root2, ((0, dp - d_mid), (0, 0)))
    bias1p = jnp.pad(bias1, (0, dp - d_mid)).reshape(1, dp)
    gamma1p = jnp.pad(gamma1, (0, dp - d_mid)).reshape(1, dp)
    beta1p = jnp.pad(beta1, (0, dp - d_mid)).reshape(1, dp)
    bias2p = bias2.reshape(1, d_out)
    gamma2p = gamma2.reshape(1, d_out)
    beta2p = beta2.reshape(1, d_out)

    zeros_d1 = jnp.zeros((n_acc, dp), F32)
    zeros_cw = jnp.zeros((n_acc, _CW), F32)
    ones_cw = jnp.ones((_CH, _CW), F32)
    zeros_d2 = jnp.zeros((n_acc, d_out), F32)

    xj1 = _sc_gather(xp, src_p)
    msg1 = _tc_messages(xj1, attr_p, w1m, b1m, d_in)
    s1, cnt = _sc_scatter(msg1, dst_p, zeros_d1, zeros_cw, ones_cw)
    x1 = _tc_combine(s1, cnt, xp, root1p, bias1p, gamma1p, beta1p,
                     n, n_acc, False)
    xj2 = _sc_gather(x1, src_p)
    msg2 = _tc_messages(xj2, attr_p, w2m, b2m, d_mid)
    s2 = _sc_scatter(msg2, dst_p, zeros_d2)
    x3 = _tc_combine(s2, cnt, x1, root2p, bias2p, gamma2p, beta2p,
                     n, n_acc, True)
    return x3


# speedup vs baseline: 6.4502x; 1.2183x over previous
import functools

import jax
import jax.numpy as jnp
from jax import lax
from jax.experimental import pallas as pl
from jax.experimental.pallas import tpu as pltpu
from jax.experimental.pallas import tpu_sc as plsc

F32 = jnp.float32
BF16 = jnp.bfloat16

_NC, _NS = 2, 16
_NW = _NC * _NS
_CH = 128
_WL = 128
_CNT = 35
_SC_PARAMS = pltpu.CompilerParams(use_tc_tiling_on_sc=False)


def _sc_gather(table, idx3):
    nw, k, ch = idx3.shape
    d = table.shape[1]
    epw = k * ch
    e_pad = nw * epw
    mesh = plsc.VectorSubcoreMesh(core_axis_name="c", subcore_axis_name="s")

    @functools.partial(
        pl.kernel,
        out_type=jax.ShapeDtypeStruct((e_pad, _WL), F32),
        mesh=mesh,
        scratch_types=[
            pltpu.VMEM((k, ch), jnp.int32),
            pltpu.VMEM((epw, d), F32),
            pltpu.SemaphoreType.DMA,
        ],
        compiler_params=_SC_PARAMS,
    )
    def kern(table_hbm, idx_hbm, out_hbm, idx_v, rows_v, sem):
        wid = lax.axis_index("s") * _NC + lax.axis_index("c")
        base = pl.multiple_of(wid * epw, ch)
        pltpu.sync_copy(idx_hbm.at[wid], idx_v)
        cps = [
            pltpu.async_copy(table_hbm.at[idx_v.at[j]],
                             rows_v.at[pl.ds(j * ch, ch)], sem)
            for j in range(k)
        ]
        for c in cps:
            c.wait()
        pltpu.sync_copy(rows_v, out_hbm.at[pl.ds(base, epw), pl.ds(0, d)])

    return kern(table, idx3)


def _sc_scatter(msgs, widths, dst3, zeros_d):
    nw, k, ch = dst3.shape
    n_acc = zeros_d.shape[0]
    rpt = n_acc // _NS
    epw = k * ch
    nm = len(msgs)
    mesh = plsc.VectorSubcoreMesh(core_axis_name="c", subcore_axis_name="s")

    out_type = tuple(jax.ShapeDtypeStruct((_NC * n_acc, _WL), F32)
                     for _ in range(nm))
    scratch = [pltpu.VMEM((k, ch), jnp.int32)]
    for w in widths:
        scratch.append(pltpu.VMEM((ch, w), F32))
        scratch.append(pltpu.VMEM_SHARED((n_acc, w), F32))

    def kern(*refs):
        msg_hbm = refs[:nm]
        dst_hbm, zd_hbm = refs[nm], refs[nm + 1]
        outs = refs[nm + 2:nm + 2 + nm]
        idx_v = refs[nm + 2 + nm]
        bufs = refs[nm + 3 + nm::2]
        accs = refs[nm + 4 + nm::2]
        cid = lax.axis_index("c")
        sid = lax.axis_index("s")
        wid = sid * _NC + cid
        base = pl.multiple_of(wid * epw, ch)
        stripe = pl.multiple_of(sid * rpt, 8)
        pltpu.sync_copy(dst_hbm.at[wid], idx_v)
        for m in range(nm):
            pltpu.sync_copy(zd_hbm.at[pl.ds(stripe, rpt), pl.ds(0, widths[m])],
                            accs[m].at[pl.ds(stripe, rpt)])
        plsc.subcore_barrier()
        for j in range(k):
            for m in range(nm):
                pltpu.sync_copy(
                    msg_hbm[m].at[pl.ds(base + j * ch, ch),
                                  pl.ds(0, widths[m])], bufs[m])
                pltpu.sync_copy(bufs[m], accs[m].at[idx_v.at[j]], add=True)
        plsc.subcore_barrier()
        obase = pl.multiple_of(cid * n_acc + stripe, 8)
        for m in range(nm):
            pltpu.sync_copy(accs[m].at[pl.ds(stripe, rpt)],
                            outs[m].at[pl.ds(obase, rpt),
                                       pl.ds(0, widths[m])])

    f = pl.kernel(kern, out_type=out_type, mesh=mesh,
                  scratch_types=scratch, compiler_params=_SC_PARAMS)
    res = f(*msgs, dst3, zeros_d)
    return list(res) if isinstance(res, (tuple, list)) else [res]


def _tc_messages(xj, attr, w_p, b_p, d_in, n_out, add_count, be=1024):
    e_pad = xj.shape[0]
    doutp = w_p.shape[1]

    def body(xj_ref, a_ref, w_ref, b_ref, *o_refs):
        a = a_ref[...].astype(BF16)
        xjb = xj_ref[...].astype(BF16)
        w = w_ref[...].astype(BF16)
        b = b_ref[...].astype(BF16)
        acc = jnp.zeros((be, doutp), BF16)
        for i in range(d_in):
            ew = jnp.maximum(a * w[i:i + 1, :] + b[i:i + 1, :], BF16(0.0))
            acc = acc + xjb[:, i:i + 1] * ew
        out = acc.astype(F32)
        if add_count:
            col = lax.broadcasted_iota(jnp.int32, (1, doutp), 1)
            out = out + jnp.where(col == _CNT, 1.0, 0.0).astype(F32)
        for m, o_ref in enumerate(o_refs):
            o_ref[...] = out[:, m * _WL:(m + 1) * _WL]

    return pl.pallas_call(
        body,
        grid=(e_pad // be,),
        in_specs=[
            pl.BlockSpec((be, _WL), lambda i: (i, 0)),
            pl.BlockSpec((be, 1), lambda i: (i, 0)),
            pl.BlockSpec(w_p.shape, lambda i: (0, 0)),
            pl.BlockSpec(b_p.shape, lambda i: (0, 0)),
        ],
        out_specs=[pl.BlockSpec((be, _WL), lambda i: (i, 0))
                   for _ in range(n_out)],
        out_shape=[jax.ShapeDtypeStruct((e_pad, _WL), F32)
                   for _ in range(n_out)],
    )(xj, attr, w_p, b_p)


def _tc_combine1(s_cat, xin, root_p, bias, gamma, beta, n, n_acc, dp):
    def body(s_ref, x_ref, r_ref, b_ref, g_ref, be_ref, o_ref):
        s = s_ref[...]
        ssum = s[:n] + s[n_acc:n_acc + n]
        cnt = jnp.maximum(ssum[:, _CNT:_CNT + 1], 1.0)
        agg = ssum[:, :dp] / cnt
        h = agg + jnp.dot(x_ref[...], r_ref[...],
                          preferred_element_type=F32) + b_ref[...]
        m = jnp.mean(h, axis=0, keepdims=True)
        v = jnp.mean((h - m) ** 2, axis=0, keepdims=True)
        z = (h - m) * lax.rsqrt(v + 1e-3) * g_ref[...] + be_ref[...]
        o_ref[...] = jax.nn.sigmoid(z)

    return pl.pallas_call(
        body,
        out_shape=jax.ShapeDtypeStruct((n, dp), F32),
    )(s_cat, xin, root_p, bias, gamma, beta)


def _tc_combine2(sa_cat, sb_cat, s1_cat, x1, root_p, bias, gamma, beta,
                 n, n_acc, d_out):
    def body(sa_ref, sb_ref, s1_ref, x_ref, r_ref, b_ref, g_ref, be_ref,
             o_ref):
        sa = sa_ref[...]
        sb = sb_ref[...]
        s1 = s1_ref[...]
        cnt = jnp.maximum(s1[:n, _CNT:_CNT + 1]
                          + s1[n_acc:n_acc + n, _CNT:_CNT + 1], 1.0)
        ha = sa[:n] + sa[n_acc:n_acc + n]
        hb = sb[:n, :d_out - _WL] + sb[n_acc:n_acc + n, :d_out - _WL]
        h = jnp.concatenate([ha, hb], axis=1) / cnt
        h = h + jnp.dot(x_ref[...], r_ref[...],
                        preferred_element_type=F32) + b_ref[...]
        m = jnp.mean(h, axis=0, keepdims=True)
        v = jnp.mean((h - m) ** 2, axis=0, keepdims=True)
        z = (h - m) * lax.rsqrt(v + 1e-3) * g_ref[...] + be_ref[...]
        x2 = jax.nn.sigmoid(z)
        o_ref[...] = lax.dot_general(x2, x2, (((0,), (0,)), ((), ())),
                                     preferred_element_type=F32)

    return pl.pallas_call(
        body,
        out_shape=jax.ShapeDtypeStruct((d_out, d_out), F32),
    )(sa_cat, sb_cat, s1_cat, x1, root_p, bias, gamma, beta)


def kernel(x, edge_index, edge_attr, w1, b1, root1, bias1, gamma1, beta1,
           w2, b2, root2, bias2, gamma2, beta2):
    n, d_in = x.shape
    e = edge_index.shape[1]
    d_mid = root1.shape[1]
    d_out = root2.shape[1]
    dp = 48
    wr = 40
    k = -(-e // (_NW * _CH))
    e_pad = _NW * _CH * k
    pad_e = e_pad - e
    n_acc = -(-(n + 1) // 128) * 128

    src = edge_index[0].astype(jnp.int32)
    dst = edge_index[1].astype(jnp.int32)
    src_p = jnp.concatenate(
        [src, jnp.zeros((pad_e,), jnp.int32)]).reshape(_NW, k, _CH)
    dst_p = jnp.concatenate(
        [dst, jnp.full((pad_e,), n, jnp.int32)]).reshape(_NW, k, _CH)
    attr_p = jnp.pad(edge_attr.astype(F32), ((0, pad_e), (0, 0)))
    xp = jnp.pad(x, ((0, 0), (0, dp - d_in)))

    w1m = jnp.pad(w1[:, 0].reshape(d_in, d_mid),
                  ((0, wr - d_in), (0, _WL - d_mid)))
    b1m = jnp.pad(b1.reshape(d_in, d_mid), ((0, wr - d_in), (0, _WL - d_mid)))
    w2m = jnp.pad(w2[:, 0].reshape(d_mid, d_out),
                  ((0, wr - d_mid), (0, 2 * _WL - d_out)))
    b2m = jnp.pad(b2.reshape(d_mid, d_out),
                  ((0, wr - d_mid), (0, 2 * _WL - d_out)))
    root1p = jnp.pad(root1, ((0, dp - d_in), (0, dp - d_mid)))
    root2p = jnp.pad(root2, ((0, dp - d_mid), (0, 0)))
    bias1p = jnp.pad(bias1, (0, dp - d_mid)).reshape(1, dp)
    gamma1p = jnp.pad(gamma1, (0, dp - d_mid)).reshape(1, dp)
    beta1p = jnp.pad(beta1, (0, dp - d_mid)).reshape(1, dp)
    bias2p = bias2.reshape(1, d_out)
    gamma2p = gamma2.reshape(1, d_out)
    beta2p = beta2.reshape(1, d_out)
    zeros_d = jnp.zeros((n_acc, _WL), F32)

    xj1 = _sc_gather(xp, src_p)
    (msg1,) = _tc_messages(xj1, attr_p, w1m, b1m, d_in, 1, True)
    (s1,) = _sc_scatter([msg1], [_WL], dst_p, zeros_d)
    x1 = _tc_combine1(s1, xp, root1p, bias1p, gamma1p, beta1p, n, n_acc, dp)
    xj2 = _sc_gather(x1, src_p)
    msg2a, msg2b = _tc_messages(xj2, attr_p, w2m, b2m, d_mid, 2, False)
    s2a, s2b = _sc_scatter([msg2a, msg2b], [_WL, d_out - _WL], dst_p, zeros_d)
    x3 = _tc_combine2(s2a, s2b, s1, x1, root2p, bias2p, gamma2p, beta2p,
                      n, n_acc, d_out)
    return x3


# speedup vs baseline: 6.6874x; 1.0368x over previous
import functools

import jax
import jax.numpy as jnp
from jax import lax
from jax.experimental import pallas as pl
from jax.experimental.pallas import tpu as pltpu
from jax.experimental.pallas import tpu_sc as plsc

F32 = jnp.float32
BF16 = jnp.bfloat16

_NC, _NS = 2, 16
_NW = _NC * _NS
_CH = 128
_WL = 128
_CNT = 35
_SC_PARAMS = pltpu.CompilerParams(use_tc_tiling_on_sc=False)


def _sc_gather(table, idx3):
    nw, k, ch = idx3.shape
    d = table.shape[1]
    epw = k * ch
    e_pad = nw * epw
    mesh = plsc.VectorSubcoreMesh(core_axis_name="c", subcore_axis_name="s")

    @functools.partial(
        pl.kernel,
        out_type=jax.ShapeDtypeStruct((e_pad, _WL), F32),
        mesh=mesh,
        scratch_types=[
            pltpu.VMEM((k, ch), jnp.int32),
            pltpu.VMEM((epw, d), F32),
            pltpu.SemaphoreType.DMA,
        ],
        compiler_params=_SC_PARAMS,
    )
    def kern(table_hbm, idx_hbm, out_hbm, idx_v, rows_v, sem):
        wid = lax.axis_index("s") * _NC + lax.axis_index("c")
        base = pl.multiple_of(wid * epw, ch)
        pltpu.sync_copy(idx_hbm.at[wid], idx_v)
        cps = [
            pltpu.async_copy(table_hbm.at[idx_v.at[j]],
                             rows_v.at[pl.ds(j * ch, ch)], sem)
            for j in range(k)
        ]
        for c in cps:
            c.wait()
        pltpu.sync_copy(rows_v, out_hbm.at[pl.ds(base, epw), pl.ds(0, d)])

    return kern(table, idx3)


def _sc_scatter(msgs, widths, dst3, zeros_d):
    nw, k, ch = dst3.shape
    n_acc = zeros_d.shape[0]
    rpt = n_acc // _NS
    epw = k * ch
    nm = len(msgs)
    mesh = plsc.VectorSubcoreMesh(core_axis_name="c", subcore_axis_name="s")

    out_type = tuple(jax.ShapeDtypeStruct((_NC * n_acc, _WL), F32)
                     for _ in range(nm))
    scratch = [pltpu.VMEM((k, ch), jnp.int32)]
    for w in widths:
        scratch.append(pltpu.VMEM((ch, w), F32))
        scratch.append(pltpu.VMEM_SHARED((n_acc, w), F32))

    def kern(*refs):
        msg_hbm = refs[:nm]
        dst_hbm, zd_hbm = refs[nm], refs[nm + 1]
        outs = refs[nm + 2:nm + 2 + nm]
        idx_v = refs[nm + 2 + nm]
        bufs = refs[nm + 3 + nm::2]
        accs = refs[nm + 4 + nm::2]
        cid = lax.axis_index("c")
        sid = lax.axis_index("s")
        wid = sid * _NC + cid
        base = pl.multiple_of(wid * epw, ch)
        stripe = pl.multiple_of(sid * rpt, 8)
        pltpu.sync_copy(dst_hbm.at[wid], idx_v)
        for m in range(nm):
            pltpu.sync_copy(zd_hbm.at[pl.ds(stripe, rpt), pl.ds(0, widths[m])],
                            accs[m].at[pl.ds(stripe, rpt)])
        plsc.subcore_barrier()
        for j in range(k):
            for m in range(nm):
                pltpu.sync_copy(
                    msg_hbm[m].at[pl.ds(base + j * ch, ch),
                                  pl.ds(0, widths[m])], bufs[m])
                pltpu.sync_copy(bufs[m], accs[m].at[idx_v.at[j]], add=True)
        plsc.subcore_barrier()
        obase = pl.multiple_of(cid * n_acc + stripe, 8)
        for m in range(nm):
            pltpu.sync_copy(accs[m].at[pl.ds(stripe, rpt)],
                            outs[m].at[pl.ds(obase, rpt),
                                       pl.ds(0, widths[m])])

    f = pl.kernel(kern, out_type=out_type, mesh=mesh,
                  scratch_types=scratch, compiler_params=_SC_PARAMS)
    res = f(*msgs, dst3, zeros_d)
    return list(res) if isinstance(res, (tuple, list)) else [res]


def _tc_messages(xj, attr, w_p, b_p, d_in, n_out, add_count, be=1024):
    e_pad = xj.shape[0]
    doutp = w_p.shape[1]

    def body(xj_ref, a_ref, w_ref, b_ref, *o_refs):
        a = jnp.reshape(a_ref[...], (be, 1)).astype(BF16)
        xjb = xj_ref[...].astype(BF16)
        w = w_ref[...].astype(BF16)
        b = b_ref[...].astype(BF16)
        acc = jnp.zeros((be, doutp), BF16)
        for i in range(d_in):
            ew = jnp.maximum(a * w[i:i + 1, :] + b[i:i + 1, :], BF16(0.0))
            acc = acc + xjb[:, i:i + 1] * ew
        out = acc.astype(F32)
        if add_count:
            col = lax.broadcasted_iota(jnp.int32, (1, doutp), 1)
            out = out + jnp.where(col == _CNT, 1.0, 0.0).astype(F32)
        for m, o_ref in enumerate(o_refs):
            o_ref[...] = out[:, m * _WL:(m + 1) * _WL]

    return pl.pallas_call(
        body,
        grid=(e_pad // be,),
        in_specs=[
            pl.BlockSpec((be, _WL), lambda i: (i, 0)),
            pl.BlockSpec((1, 1, be), lambda i: (i, 0, 0)),
            pl.BlockSpec(w_p.shape, lambda i: (0, 0)),
            pl.BlockSpec(b_p.shape, lambda i: (0, 0)),
        ],
        out_specs=[pl.BlockSpec((be, _WL), lambda i: (i, 0))
                   for _ in range(n_out)],
        out_shape=[jax.ShapeDtypeStruct((e_pad, _WL), F32)
                   for _ in range(n_out)],
    )(xj, attr, w_p, b_p)


def _tc_combine1(s_cat, xin, root_p, bias, gamma, beta, n, n_acc, dp):
    def body(s_ref, x_ref, r_ref, b_ref, g_ref, be_ref, o_ref):
        s = s_ref[...]
        ssum = s[:n] + s[n_acc:n_acc + n]
        cnt = jnp.maximum(ssum[:, _CNT:_CNT + 1], 1.0)
        agg = ssum[:, :dp] / cnt
        h = agg + jnp.dot(x_ref[...], r_ref[...],
                          preferred_element_type=F32) + b_ref[...]
        m = jnp.mean(h, axis=0, keepdims=True)
        v = jnp.mean((h - m) ** 2, axis=0, keepdims=True)
        z = (h - m) * lax.rsqrt(v + 1e-3) * g_ref[...] + be_ref[...]
        x1v = jax.nn.sigmoid(z)
        col = lax.broadcasted_iota(jnp.int32, (1, dp), 1)
        o_ref[...] = jnp.where(col == _CNT, cnt, x1v)

    return pl.pallas_call(
        body,
        out_shape=jax.ShapeDtypeStruct((n, dp), F32),
    )(s_cat, xin, root_p, bias, gamma, beta)


def _tc_combine2(sa_cat, sb_cat, x1, root_p, bias, gamma, beta,
                 n, n_acc, d_out):
    def body(sa_ref, sb_ref, x_ref, r_ref, b_ref, g_ref, be_ref,
             o_ref):
        sa = sa_ref[...]
        sb = sb_ref[...]
        cnt = x_ref[:, _CNT:_CNT + 1]
        ha = sa[:n] + sa[n_acc:n_acc + n]
        hb = sb[:n, :d_out - _WL] + sb[n_acc:n_acc + n, :d_out - _WL]
        h = jnp.concatenate([ha, hb], axis=1) / cnt
        h = h + jnp.dot(x_ref[...], r_ref[...],
                        preferred_element_type=F32) + b_ref[...]
        m = jnp.mean(h, axis=0, keepdims=True)
        v = jnp.mean((h - m) ** 2, axis=0, keepdims=True)
        z = (h - m) * lax.rsqrt(v + 1e-3) * g_ref[...] + be_ref[...]
        x2 = jax.nn.sigmoid(z)
        o_ref[...] = lax.dot_general(x2, x2, (((0,), (0,)), ((), ())),
                                     preferred_element_type=F32)

    return pl.pallas_call(
        body,
        out_shape=jax.ShapeDtypeStruct((d_out, d_out), F32),
    )(sa_cat, sb_cat, x1, root_p, bias, gamma, beta)


def kernel(x, edge_index, edge_attr, w1, b1, root1, bias1, gamma1, beta1,
           w2, b2, root2, bias2, gamma2, beta2):
    n, d_in = x.shape
    e = edge_index.shape[1]
    d_mid = root1.shape[1]
    d_out = root2.shape[1]
    dp = 48
    wr = 40
    k = -(-e // (_NW * _CH))
    e_pad = _NW * _CH * k
    pad_e = e_pad - e
    n_acc = -(-(n + 1) // 128) * 128

    src = edge_index[0].astype(jnp.int32)
    dst = edge_index[1].astype(jnp.int32)
    src_p = jnp.concatenate(
        [src, jnp.zeros((pad_e,), jnp.int32)]).reshape(_NW, k, _CH)
    dst_p = jnp.concatenate(
        [dst, jnp.full((pad_e,), n, jnp.int32)]).reshape(_NW, k, _CH)
    attr_p = jnp.pad(edge_attr.astype(F32)[:, 0],
                     (0, pad_e)).reshape(-1, 1, 1024)
    xp = jnp.pad(x, ((0, 0), (0, dp - d_in)))

    w1m = jnp.pad(w1[:, 0].reshape(d_in, d_mid),
                  ((0, wr - d_in), (0, _WL - d_mid)))
    b1m = jnp.pad(b1.reshape(d_in, d_mid), ((0, wr - d_in), (0, _WL - d_mid)))
    w2m = jnp.pad(w2[:, 0].reshape(d_mid, d_out),
                  ((0, wr - d_mid), (0, 2 * _WL - d_out)))
    b2m = jnp.pad(b2.reshape(d_mid, d_out),
                  ((0, wr - d_mid), (0, 2 * _WL - d_out)))
    root1p = jnp.pad(root1, ((0, dp - d_in), (0, dp - d_mid)))
    root2p = jnp.pad(root2, ((0, dp - d_mid), (0, 0)))
    bias1p = jnp.pad(bias1, (0, dp - d_mid)).reshape(1, dp)
    gamma1p = jnp.pad(gamma1, (0, dp - d_mid)).reshape(1, dp)
    beta1p = jnp.pad(beta1, (0, dp - d_mid)).reshape(1, dp)
    bias2p = bias2.reshape(1, d_out)
    gamma2p = gamma2.reshape(1, d_out)
    beta2p = beta2.reshape(1, d_out)
    zeros_d = jnp.zeros((n_acc, _WL), F32)

    xj1 = _sc_gather(xp, src_p)
    (msg1,) = _tc_messages(xj1, attr_p, w1m, b1m, d_in, 1, True)
    (s1,) = _sc_scatter([msg1], [_WL], dst_p, zeros_d)
    x1 = _tc_combine1(s1, xp, root1p, bias1p, gamma1p, beta1p, n, n_acc, dp)
    xj2 = _sc_gather(x1, src_p)
    msg2a, msg2b = _tc_messages(xj2, attr_p, w2m, b2m, d_mid, 2, False)
    s2a, s2b = _sc_scatter([msg2a, msg2b], [_WL, d_out - _WL], dst_p, zeros_d)
    x3 = _tc_combine2(s2a, s2b, x1, root2p, bias2p, gamma2p, beta2p,
                      n, n_acc, d_out)
    return x3
